# baseline (speedup 1.0000x reference)
import dataclasses
import functools

import jax
import jax.numpy as jnp
from jax import lax
from jax.experimental import pallas as pl
from jax.experimental.pallas import tpu as pltpu
from jax.experimental.pallas import tpu_sc as plsc

N = 10000
E = 320000
D = 128
NCORE = 2
NSUB = 16
LANES = 16
CHUNK = 64
NBUF = 2
EPT = E // NSUB
NCHUNK = 320
EPT_PAD = NCHUNK * CHUNK
BATCH = 40
NBATCH = NCHUNK // BATCH
ROWS_PT = 632
ROWS_LAST = N - 15 * ROWS_PT


def _scale_rows(rows, wv, c):
    widx_c = jnp.full((LANES,), 0, jnp.int32) + c

    @pl.loop(0, CHUNK)
    def _(e):
        widx_e = jnp.full((LANES,), 0, jnp.int32) + e
        w = plsc.load_gather(wv, [widx_c, widx_e])
        for j in range(D // LANES):
            sl = (e, pl.ds(j * LANES, LANES))
            rows[sl] = rows[sl] * w


def _seg_body(x_hbm, src_hbm, dst_hbm, w_hbm, zero_hbm, out_hbm,
              srcv, dstv, wv, r0, r1, acc, sg0, sg1, ss0, ss1):
    rows = [r0, r1]
    sg = [sg0, sg1]
    ss = [ss0, ss1]
    cid = lax.axis_index("c")
    sid = lax.axis_index("s")
    row0 = sid * ROWS_PT
    @pl.when(sid < NSUB - 1)
    def _():
        pltpu.sync_copy(zero_hbm.at[pl.ds(row0, ROWS_PT)],
                        acc.at[pl.ds(row0, ROWS_PT)])

    @pl.when(sid == NSUB - 1)
    def _():
        pltpu.sync_copy(zero_hbm.at[pl.ds(15 * ROWS_PT, ROWS_LAST)],
                        acc.at[pl.ds(15 * ROWS_PT, ROWS_LAST)])

    plsc.subcore_barrier()

    @pl.loop(0, NBATCH)
    def _(b):
        pltpu.sync_copy(src_hbm.at[cid, sid, b], srcv)
        pltpu.sync_copy(dst_hbm.at[cid, sid, b], dstv)
        pltpu.sync_copy(w_hbm.at[cid, sid, b], wv)

        for k in range(NBUF):
            pltpu.async_copy(x_hbm.at[srcv.at[k]], rows[k], sg[k])

        @pl.loop(0, BATCH, step=NBUF)
        def _(c):
            for k in range(NBUF):
                ck = c + k
                pltpu.make_async_copy(x_hbm.at[srcv.at[ck]], rows[k],
                                      sg[k]).wait()
                _scale_rows(rows[k], wv, ck)
                pltpu.async_copy(rows[k], acc.at[dstv.at[ck]], ss[k],
                                 add=True)

            for k in range(NBUF):
                ck = c + k
                pltpu.make_async_copy(rows[k], acc.at[dstv.at[ck]],
                                      ss[k]).wait()

                @pl.when(ck + NBUF < BATCH)
                def _():
                    pltpu.async_copy(x_hbm.at[srcv.at[ck + NBUF]], rows[k],
                                     sg[k])

    plsc.subcore_barrier()

    @pl.when(sid < NSUB - 1)
    def _():
        pltpu.sync_copy(acc.at[pl.ds(row0, ROWS_PT)],
                        out_hbm.at[cid, pl.ds(row0, ROWS_PT)])

    @pl.when(sid == NSUB - 1)
    def _():
        pltpu.sync_copy(acc.at[pl.ds(15 * ROWS_PT, ROWS_LAST)],
                        out_hbm.at[cid, pl.ds(15 * ROWS_PT, ROWS_LAST)])


def _sc_aggregate(x, src2, dst2, w2, zeros):
    mesh = plsc.VectorSubcoreMesh(core_axis_name="c", subcore_axis_name="s")
    cp = pltpu.CompilerParams()
    if "needs_layout_passes" in pltpu.CompilerParams.__dataclass_fields__:
        cp = dataclasses.replace(cp, needs_layout_passes=False)
    kern = pl.kernel(
        _seg_body,
        out_type=jax.ShapeDtypeStruct((NCORE, N, D), jnp.float32),
        mesh=mesh,
        scratch_types=[
            pltpu.VMEM((BATCH, CHUNK), jnp.int32),
            pltpu.VMEM((BATCH, CHUNK), jnp.int32),
            pltpu.VMEM((BATCH, CHUNK), jnp.float32),
            pltpu.VMEM((CHUNK, D), jnp.float32),
            pltpu.VMEM((CHUNK, D), jnp.float32),
            pltpu.VMEM_SHARED((N, D), jnp.float32),
            pltpu.SemaphoreType.DMA,
            pltpu.SemaphoreType.DMA,
            pltpu.SemaphoreType.DMA,
            pltpu.SemaphoreType.DMA,
        ],
        compiler_params=cp,
    )
    return kern(x, src2, dst2, w2, zeros)


RB = 2000


def _lin_body(aL_ref, aN_ref, Wc_ref, Wh_ref, Wl_ref, bc_ref, bh_ref, bl_ref,
              o_ref):
    h = jnp.dot(aL_ref[...], Wc_ref[...], preferred_element_type=jnp.float32)
    h = h + 0.5 * jnp.dot(aN_ref[...], Wh_ref[...],
                          preferred_element_type=jnp.float32)
    h = h + (bc_ref[...] + 0.5 * bh_ref[...])
    o_ref[...] = (jnp.dot(h, Wl_ref[...], preferred_element_type=jnp.float32)
                  + bl_ref[...])


def _linear(aggL, aggN, Wc, Wh, Wl, bc, bh, bl):
    full = lambda i: (0, 0)
    return pl.pallas_call(
        _lin_body,
        grid=(N // RB,),
        in_specs=[
            pl.BlockSpec((RB, D), lambda i: (i, 0)),
            pl.BlockSpec((RB, D), lambda i: (i, 0)),
            pl.BlockSpec((D, D), full),
            pl.BlockSpec((D, D), full),
            pl.BlockSpec((D, D), full),
            pl.BlockSpec((1, D), full),
            pl.BlockSpec((1, D), full),
            pl.BlockSpec((1, D), full),
        ],
        out_specs=pl.BlockSpec((RB, D), lambda i: (i, 0)),
        out_shape=jax.ShapeDtypeStruct((N, D), jnp.float32),
    )(aggL, aggN, Wc, Wh, Wl, bc, bh, bl)


def kernel(x, edge_index_low, edge_weight_low, edge_index_high,
           edge_weight_high, edge_index_nd_low, edge_weight_nd_low,
           edge_index_nd_high, edge_weight_nd_high,
           W_conv, b_conv, W_hiconv, b_hiconv, W_lin, b_lin):
    pad = EPT_PAD - EPT

    def _prep(a, fill):
        a = a.reshape(NCORE, NSUB, EPT)
        a = jnp.pad(a, ((0, 0), (0, 0), (0, pad)), constant_values=fill)
        return a.reshape(NCORE, NSUB, NBATCH, BATCH, CHUNK)

    src2 = _prep(jnp.stack([edge_index_low[0], edge_index_nd_low[0]]), 0)
    dst2 = _prep(jnp.stack([edge_index_low[1], edge_index_nd_low[1]]), 0)
    w2 = _prep(jnp.stack([edge_weight_low, edge_weight_nd_low]), 0.0)
    zeros = jnp.zeros((N, D), jnp.float32)

    agg = _sc_aggregate(x, src2, dst2, w2, zeros)
    return _linear(agg[0], agg[1], W_conv, W_hiconv, W_lin,
                   b_conv.reshape(1, D), b_hiconv.reshape(1, D),
                   b_lin.reshape(1, D))


# speedup vs baseline: 2.1700x; 2.1700x over previous
import dataclasses
import functools

import jax
import jax.numpy as jnp
from jax import lax
from jax.experimental import pallas as pl
from jax.experimental.pallas import tpu as pltpu
from jax.experimental.pallas import tpu_sc as plsc

N = 10000
E = 320000
D = 128
NCORE = 2
NSUB = 16
LANES = 16
CHUNK = 64
NBUF = 2
EPT = E // NSUB
NCHUNK = 320
EPT_PAD = NCHUNK * CHUNK
BATCH = 40
NBATCH = NCHUNK // BATCH
ROWS_PT = 632
ROWS_LAST = N - 15 * ROWS_PT


def _scale_rows(rows, wv, c):
    widx_c = jnp.full((LANES,), 0, jnp.int32) + c

    @pl.loop(0, CHUNK)
    def _(e):
        widx_e = jnp.full((LANES,), 0, jnp.int32) + e
        w = plsc.load_gather(wv, [widx_c, widx_e])
        for j in range(D // LANES):
            sl = (e, pl.ds(j * LANES, LANES))
            rows[sl] = rows[sl] * w


def _seg_body(x_hbm, src_hbm, dst_hbm, w_hbm, zero_hbm, out_hbm,
              srcv, dstv, wv, r0, r1, acc, sg0, sg1, ss0, ss1):
    rows = [r0, r1]
    sg = [sg0, sg1]
    ss = [ss0, ss1]
    cid = lax.axis_index("c")
    sid = lax.axis_index("s")
    row0 = sid * ROWS_PT
    @pl.when(sid < NSUB - 1)
    def _():
        pltpu.sync_copy(zero_hbm.at[pl.ds(row0, ROWS_PT)],
                        acc.at[pl.ds(row0, ROWS_PT)])

    @pl.when(sid == NSUB - 1)
    def _():
        pltpu.sync_copy(zero_hbm.at[pl.ds(15 * ROWS_PT, ROWS_LAST)],
                        acc.at[pl.ds(15 * ROWS_PT, ROWS_LAST)])

    plsc.subcore_barrier()

    @pl.loop(0, NBATCH)
    def _(b):
        pltpu.sync_copy(src_hbm.at[cid, sid, b], srcv)
        pltpu.sync_copy(dst_hbm.at[cid, sid, b], dstv)
        pltpu.sync_copy(w_hbm.at[cid, sid, b], wv)

        for k in range(NBUF):
            pltpu.async_copy(x_hbm.at[srcv.at[k]], rows[k], sg[k])

        @pl.loop(0, BATCH, step=NBUF)
        def _(c):
            for k in range(NBUF):
                ck = c + k
                pltpu.make_async_copy(x_hbm.at[srcv.at[ck]], rows[k],
                                      sg[k]).wait()
                _scale_rows(rows[k], wv, ck)
                pltpu.async_copy(rows[k], acc.at[dstv.at[ck]], ss[k],
                                 add=True)

            for k in range(NBUF):
                ck = c + k
                pltpu.make_async_copy(rows[k], acc.at[dstv.at[ck]],
                                      ss[k]).wait()

                @pl.when(ck + NBUF < BATCH)
                def _():
                    pltpu.async_copy(x_hbm.at[srcv.at[ck + NBUF]], rows[k],
                                     sg[k])

    plsc.subcore_barrier()

    @pl.when(sid < NSUB - 1)
    def _():
        pltpu.sync_copy(acc.at[pl.ds(row0, ROWS_PT)],
                        out_hbm.at[cid, pl.ds(row0, ROWS_PT)])

    @pl.when(sid == NSUB - 1)
    def _():
        pltpu.sync_copy(acc.at[pl.ds(15 * ROWS_PT, ROWS_LAST)],
                        out_hbm.at[cid, pl.ds(15 * ROWS_PT, ROWS_LAST)])


def _sc_aggregate(x, src2, dst2, w2, zeros):
    mesh = plsc.VectorSubcoreMesh(core_axis_name="c", subcore_axis_name="s")
    cp = pltpu.CompilerParams()
    if "needs_layout_passes" in pltpu.CompilerParams.__dataclass_fields__:
        cp = dataclasses.replace(cp, needs_layout_passes=False)
    kern = pl.kernel(
        _seg_body,
        out_type=jax.ShapeDtypeStruct((NCORE, N, D), jnp.float32),
        mesh=mesh,
        scratch_types=[
            pltpu.VMEM((BATCH, CHUNK), jnp.int32),
            pltpu.VMEM((BATCH, CHUNK), jnp.int32),
            pltpu.VMEM((BATCH, CHUNK), jnp.float32),
            pltpu.VMEM((CHUNK, D), jnp.float32),
            pltpu.VMEM((CHUNK, D), jnp.float32),
            pltpu.VMEM_SHARED((N, D), jnp.float32),
            pltpu.SemaphoreType.DMA,
            pltpu.SemaphoreType.DMA,
            pltpu.SemaphoreType.DMA,
            pltpu.SemaphoreType.DMA,
        ],
        compiler_params=cp,
    )
    return kern(x, src2, dst2, w2, zeros)


RB = 2000


def _lin_body(aL_ref, aN_ref, Wc_ref, Wh_ref, Wl_ref, bc_ref, bh_ref, bl_ref,
              o_ref):
    h = jnp.dot(aL_ref[...], Wc_ref[...], preferred_element_type=jnp.float32)
    h = h + 0.5 * jnp.dot(aN_ref[...], Wh_ref[...],
                          preferred_element_type=jnp.float32)
    h = h + (bc_ref[...] + 0.5 * bh_ref[...])
    o_ref[...] = (jnp.dot(h, Wl_ref[...], preferred_element_type=jnp.float32)
                  + bl_ref[...])


def _linear(aggL, aggN, Wc, Wh, Wl, bc, bh, bl):
    full = lambda i: (0, 0)
    return pl.pallas_call(
        _lin_body,
        grid=(N // RB,),
        in_specs=[
            pl.BlockSpec((RB, D), lambda i: (i, 0)),
            pl.BlockSpec((RB, D), lambda i: (i, 0)),
            pl.BlockSpec((D, D), full),
            pl.BlockSpec((D, D), full),
            pl.BlockSpec((D, D), full),
            pl.BlockSpec((1, D), full),
            pl.BlockSpec((1, D), full),
            pl.BlockSpec((1, D), full),
        ],
        out_specs=pl.BlockSpec((RB, D), lambda i: (i, 0)),
        out_shape=jax.ShapeDtypeStruct((N, D), jnp.float32),
    )(aggL, aggN, Wc, Wh, Wl, bc, bh, bl)


def kernel(x, edge_index_low, edge_weight_low, edge_index_high,
           edge_weight_high, edge_index_nd_low, edge_weight_nd_low,
           edge_index_nd_high, edge_weight_nd_high,
           W_conv, b_conv, W_hiconv, b_hiconv, W_lin, b_lin):
    pad = EPT_PAD - EPT
    spread = ((jnp.arange(pad, dtype=jnp.int32)[None, None, :] * 131
               + 613 * jnp.arange(NSUB, dtype=jnp.int32)[None, :, None])
              % N) + jnp.zeros((NCORE, 1, 1), jnp.int32)

    def _prep_idx(a):
        a = a.reshape(NCORE, NSUB, EPT)
        a = jnp.concatenate([a, spread], axis=2)
        return a.reshape(NCORE, NSUB, NBATCH, BATCH, CHUNK)

    def _prep_w(a):
        a = a.reshape(NCORE, NSUB, EPT)
        a = jnp.pad(a, ((0, 0), (0, 0), (0, pad)))
        return a.reshape(NCORE, NSUB, NBATCH, BATCH, CHUNK)

    src2 = _prep_idx(jnp.stack([edge_index_low[0], edge_index_nd_low[0]]))
    dst2 = _prep_idx(jnp.stack([edge_index_low[1], edge_index_nd_low[1]]))
    w2 = _prep_w(jnp.stack([edge_weight_low, edge_weight_nd_low]))
    zeros = jnp.zeros((N, D), jnp.float32)

    agg = _sc_aggregate(x, src2, dst2, w2, zeros)
    return _linear(agg[0], agg[1], W_conv, W_hiconv, W_lin,
                   b_conv.reshape(1, D), b_hiconv.reshape(1, D),
                   b_lin.reshape(1, D))


# speedup vs baseline: 2.2914x; 1.0559x over previous
import dataclasses
import functools

import jax
import jax.numpy as jnp
from jax import lax
from jax.experimental import pallas as pl
from jax.experimental.pallas import tpu as pltpu
from jax.experimental.pallas import tpu_sc as plsc

N = 10000
E = 320000
D = 128
NCORE = 2
NSUB = 16
LANES = 16
CHUNK = 128
NBUF = 2
EPT = E // NSUB
NCHUNK = 160
EPT_PAD = NCHUNK * CHUNK
BATCH = 8
NBATCH = NCHUNK // BATCH
ROWS_PT = 632
ROWS_LAST = N - 15 * ROWS_PT


def _scale_rows(rows, wv, c):
    widx_c = jnp.full((LANES,), 0, jnp.int32) + c

    @pl.loop(0, CHUNK)
    def _(e):
        widx_e = jnp.full((LANES,), 0, jnp.int32) + e
        w = plsc.load_gather(wv, [widx_c, widx_e])
        for j in range(D // LANES):
            sl = (e, pl.ds(j * LANES, LANES))
            rows[sl] = rows[sl] * w


def _seg_body(x_hbm, src_hbm, dst_hbm, w_hbm, zero_hbm, out_hbm,
              srcv, dstv, wv, r0, r1, acc, sg0, sg1, ss0, ss1):
    rows = [r0, r1]
    sg = [sg0, sg1]
    ss = [ss0, ss1]
    cid = lax.axis_index("c")
    sid = lax.axis_index("s")
    row0 = sid * ROWS_PT
    @pl.when(sid < NSUB - 1)
    def _():
        pltpu.sync_copy(zero_hbm.at[pl.ds(row0, ROWS_PT)],
                        acc.at[pl.ds(row0, ROWS_PT)])

    @pl.when(sid == NSUB - 1)
    def _():
        pltpu.sync_copy(zero_hbm.at[pl.ds(15 * ROWS_PT, ROWS_LAST)],
                        acc.at[pl.ds(15 * ROWS_PT, ROWS_LAST)])

    plsc.subcore_barrier()

    @pl.loop(0, NBATCH)
    def _(b):
        pltpu.sync_copy(src_hbm.at[cid, sid, b], srcv)
        pltpu.sync_copy(dst_hbm.at[cid, sid, b], dstv)
        pltpu.sync_copy(w_hbm.at[cid, sid, b], wv)

        for k in range(NBUF):
            pltpu.async_copy(x_hbm.at[srcv.at[k]], rows[k], sg[k])

        @pl.loop(0, BATCH, step=NBUF)
        def _(c):
            for k in range(NBUF):
                ck = c + k
                pltpu.make_async_copy(x_hbm.at[srcv.at[ck]], rows[k],
                                      sg[k]).wait()
                _scale_rows(rows[k], wv, ck)
                pltpu.async_copy(rows[k], acc.at[dstv.at[ck]], ss[k],
                                 add=True)

            for k in range(NBUF):
                ck = c + k
                pltpu.make_async_copy(rows[k], acc.at[dstv.at[ck]],
                                      ss[k]).wait()

                @pl.when(ck + NBUF < BATCH)
                def _():
                    pltpu.async_copy(x_hbm.at[srcv.at[ck + NBUF]], rows[k],
                                     sg[k])

    plsc.subcore_barrier()

    @pl.when(sid < NSUB - 1)
    def _():
        pltpu.sync_copy(acc.at[pl.ds(row0, ROWS_PT)],
                        out_hbm.at[cid, pl.ds(row0, ROWS_PT)])

    @pl.when(sid == NSUB - 1)
    def _():
        pltpu.sync_copy(acc.at[pl.ds(15 * ROWS_PT, ROWS_LAST)],
                        out_hbm.at[cid, pl.ds(15 * ROWS_PT, ROWS_LAST)])


def _sc_aggregate(x, src2, dst2, w2, zeros):
    mesh = plsc.VectorSubcoreMesh(core_axis_name="c", subcore_axis_name="s")
    cp = pltpu.CompilerParams()
    if "needs_layout_passes" in pltpu.CompilerParams.__dataclass_fields__:
        cp = dataclasses.replace(cp, needs_layout_passes=False)
    kern = pl.kernel(
        _seg_body,
        out_type=jax.ShapeDtypeStruct((NCORE, N, D), jnp.float32),
        mesh=mesh,
        scratch_types=[
            pltpu.VMEM((BATCH, CHUNK), jnp.int32),
            pltpu.VMEM((BATCH, CHUNK), jnp.int32),
            pltpu.VMEM((BATCH, CHUNK), jnp.float32),
            pltpu.VMEM((CHUNK, D), jnp.float32),
            pltpu.VMEM((CHUNK, D), jnp.float32),
            pltpu.VMEM_SHARED((N, D), jnp.float32),
            pltpu.SemaphoreType.DMA,
            pltpu.SemaphoreType.DMA,
            pltpu.SemaphoreType.DMA,
            pltpu.SemaphoreType.DMA,
        ],
        compiler_params=cp,
    )
    return kern(x, src2, dst2, w2, zeros)


RB = 2000


def _lin_body(aL_ref, aN_ref, Wc_ref, Wh_ref, Wl_ref, bc_ref, bh_ref, bl_ref,
              o_ref):
    h = jnp.dot(aL_ref[...], Wc_ref[...], preferred_element_type=jnp.float32)
    h = h + 0.5 * jnp.dot(aN_ref[...], Wh_ref[...],
                          preferred_element_type=jnp.float32)
    h = h + (bc_ref[...] + 0.5 * bh_ref[...])
    o_ref[...] = (jnp.dot(h, Wl_ref[...], preferred_element_type=jnp.float32)
                  + bl_ref[...])


def _linear(aggL, aggN, Wc, Wh, Wl, bc, bh, bl):
    full = lambda i: (0, 0)
    return pl.pallas_call(
        _lin_body,
        grid=(N // RB,),
        in_specs=[
            pl.BlockSpec((RB, D), lambda i: (i, 0)),
            pl.BlockSpec((RB, D), lambda i: (i, 0)),
            pl.BlockSpec((D, D), full),
            pl.BlockSpec((D, D), full),
            pl.BlockSpec((D, D), full),
            pl.BlockSpec((1, D), full),
            pl.BlockSpec((1, D), full),
            pl.BlockSpec((1, D), full),
        ],
        out_specs=pl.BlockSpec((RB, D), lambda i: (i, 0)),
        out_shape=jax.ShapeDtypeStruct((N, D), jnp.float32),
    )(aggL, aggN, Wc, Wh, Wl, bc, bh, bl)


def kernel(x, edge_index_low, edge_weight_low, edge_index_high,
           edge_weight_high, edge_index_nd_low, edge_weight_nd_low,
           edge_index_nd_high, edge_weight_nd_high,
           W_conv, b_conv, W_hiconv, b_hiconv, W_lin, b_lin):
    pad = EPT_PAD - EPT
    spread = ((jnp.arange(pad, dtype=jnp.int32)[None, None, :] * 131
               + 613 * jnp.arange(NSUB, dtype=jnp.int32)[None, :, None])
              % N) + jnp.zeros((NCORE, 1, 1), jnp.int32)

    def _prep_idx(a):
        a = a.reshape(NCORE, NSUB, EPT)
        a = jnp.concatenate([a, spread], axis=2)
        return a.reshape(NCORE, NSUB, NBATCH, BATCH, CHUNK)

    def _prep_w(a):
        a = a.reshape(NCORE, NSUB, EPT)
        a = jnp.pad(a, ((0, 0), (0, 0), (0, pad)))
        return a.reshape(NCORE, NSUB, NBATCH, BATCH, CHUNK)

    src2 = _prep_idx(jnp.stack([edge_index_low[0], edge_index_nd_low[0]]))
    dst2 = _prep_idx(jnp.stack([edge_index_low[1], edge_index_nd_low[1]]))
    w2 = _prep_w(jnp.stack([edge_weight_low, edge_weight_nd_low]))
    zeros = jnp.zeros((N, D), jnp.float32)

    agg = _sc_aggregate(x, src2, dst2, w2, zeros)
    return _linear(agg[0], agg[1], W_conv, W_hiconv, W_lin,
                   b_conv.reshape(1, D), b_hiconv.reshape(1, D),
                   b_lin.reshape(1, D))


# speedup vs baseline: 2.4872x; 1.0855x over previous
import dataclasses
import functools

import jax
import jax.numpy as jnp
from jax import lax
from jax.experimental import pallas as pl
from jax.experimental.pallas import tpu as pltpu
from jax.experimental.pallas import tpu_sc as plsc

N = 10000
E = 320000
D = 128
NCORE = 2
NSUB = 16
LANES = 16
CHUNK = 80
NBUF = 2
EPT = E // NSUB
NCHUNK = 250
EPT_PAD = NCHUNK * CHUNK
BATCH = 50
NBATCH = NCHUNK // BATCH
ROWS_PT = 632
ROWS_LAST = N - 15 * ROWS_PT


def _scale_rows(rows, wv, c):
    widx_c = jnp.full((LANES,), 0, jnp.int32) + c

    @pl.loop(0, CHUNK, unroll=4)
    def _(e):
        widx_e = jnp.full((LANES,), 0, jnp.int32) + e
        w = plsc.load_gather(wv, [widx_c, widx_e])
        for j in range(D // LANES):
            sl = (e, pl.ds(j * LANES, LANES))
            rows[sl] = rows[sl] * w


def _seg_body(x_hbm, src_hbm, dst_hbm, w_hbm, zero_hbm, out_hbm,
              srcv, dstv, wv, r0, r1, acc, sg0, sg1, ss0, ss1):
    rows = [r0, r1]
    sg = [sg0, sg1]
    ss = [ss0, ss1]
    cid = lax.axis_index("c")
    sid = lax.axis_index("s")
    row0 = sid * ROWS_PT
    @pl.when(sid < NSUB - 1)
    def _():
        pltpu.sync_copy(zero_hbm.at[pl.ds(row0, ROWS_PT)],
                        acc.at[pl.ds(row0, ROWS_PT)])

    @pl.when(sid == NSUB - 1)
    def _():
        pltpu.sync_copy(zero_hbm.at[pl.ds(15 * ROWS_PT, ROWS_LAST)],
                        acc.at[pl.ds(15 * ROWS_PT, ROWS_LAST)])

    plsc.subcore_barrier()

    @pl.loop(0, NBATCH)
    def _(b):
        pltpu.sync_copy(src_hbm.at[cid, sid, b], srcv)
        pltpu.sync_copy(dst_hbm.at[cid, sid, b], dstv)
        pltpu.sync_copy(w_hbm.at[cid, sid, b], wv)

        for k in range(NBUF):
            pltpu.async_copy(x_hbm.at[srcv.at[k]], rows[k], sg[k])

        @pl.loop(0, BATCH, step=NBUF)
        def _(c):
            for k in range(NBUF):
                ck = c + k
                pltpu.make_async_copy(x_hbm.at[srcv.at[ck]], rows[k],
                                      sg[k]).wait()
                _scale_rows(rows[k], wv, ck)
                pltpu.async_copy(rows[k], acc.at[dstv.at[ck]], ss[k],
                                 add=True)

            for k in range(NBUF):
                ck = c + k
                pltpu.make_async_copy(rows[k], acc.at[dstv.at[ck]],
                                      ss[k]).wait()

                @pl.when(ck + NBUF < BATCH)
                def _():
                    pltpu.async_copy(x_hbm.at[srcv.at[ck + NBUF]], rows[k],
                                     sg[k])

    plsc.subcore_barrier()

    @pl.when(sid < NSUB - 1)
    def _():
        pltpu.sync_copy(acc.at[pl.ds(row0, ROWS_PT)],
                        out_hbm.at[cid, pl.ds(row0, ROWS_PT)])

    @pl.when(sid == NSUB - 1)
    def _():
        pltpu.sync_copy(acc.at[pl.ds(15 * ROWS_PT, ROWS_LAST)],
                        out_hbm.at[cid, pl.ds(15 * ROWS_PT, ROWS_LAST)])


def _sc_aggregate(x, src2, dst2, w2, zeros):
    mesh = plsc.VectorSubcoreMesh(core_axis_name="c", subcore_axis_name="s")
    cp = pltpu.CompilerParams()
    if "needs_layout_passes" in pltpu.CompilerParams.__dataclass_fields__:
        cp = dataclasses.replace(cp, needs_layout_passes=False)
    kern = pl.kernel(
        _seg_body,
        out_type=jax.ShapeDtypeStruct((NCORE, N, D), jnp.float32),
        mesh=mesh,
        scratch_types=[
            pltpu.VMEM((BATCH, CHUNK), jnp.int32),
            pltpu.VMEM((BATCH, CHUNK), jnp.int32),
            pltpu.VMEM((BATCH, CHUNK), jnp.float32),
            pltpu.VMEM((CHUNK, D), jnp.float32),
            pltpu.VMEM((CHUNK, D), jnp.float32),
            pltpu.VMEM_SHARED((N, D), jnp.float32),
            pltpu.SemaphoreType.DMA,
            pltpu.SemaphoreType.DMA,
            pltpu.SemaphoreType.DMA,
            pltpu.SemaphoreType.DMA,
        ],
        compiler_params=cp,
    )
    return kern(x, src2, dst2, w2, zeros)


RB = 2000


def _lin_body(aL_ref, aN_ref, Wc_ref, Wh_ref, Wl_ref, bc_ref, bh_ref, bl_ref,
              o_ref):
    h = jnp.dot(aL_ref[...], Wc_ref[...], preferred_element_type=jnp.float32)
    h = h + 0.5 * jnp.dot(aN_ref[...], Wh_ref[...],
                          preferred_element_type=jnp.float32)
    h = h + (bc_ref[...] + 0.5 * bh_ref[...])
    o_ref[...] = (jnp.dot(h, Wl_ref[...], preferred_element_type=jnp.float32)
                  + bl_ref[...])


def _linear(aggL, aggN, Wc, Wh, Wl, bc, bh, bl):
    full = lambda i: (0, 0)
    return pl.pallas_call(
        _lin_body,
        grid=(N // RB,),
        in_specs=[
            pl.BlockSpec((RB, D), lambda i: (i, 0)),
            pl.BlockSpec((RB, D), lambda i: (i, 0)),
            pl.BlockSpec((D, D), full),
            pl.BlockSpec((D, D), full),
            pl.BlockSpec((D, D), full),
            pl.BlockSpec((1, D), full),
            pl.BlockSpec((1, D), full),
            pl.BlockSpec((1, D), full),
        ],
        out_specs=pl.BlockSpec((RB, D), lambda i: (i, 0)),
        out_shape=jax.ShapeDtypeStruct((N, D), jnp.float32),
    )(aggL, aggN, Wc, Wh, Wl, bc, bh, bl)


def kernel(x, edge_index_low, edge_weight_low, edge_index_high,
           edge_weight_high, edge_index_nd_low, edge_weight_nd_low,
           edge_index_nd_high, edge_weight_nd_high,
           W_conv, b_conv, W_hiconv, b_hiconv, W_lin, b_lin):
    pad = EPT_PAD - EPT
    spread = ((jnp.arange(pad, dtype=jnp.int32)[None, None, :] * 131
               + 613 * jnp.arange(NSUB, dtype=jnp.int32)[None, :, None])
              % N) + jnp.zeros((NCORE, 1, 1), jnp.int32)

    def _prep_idx(a):
        a = a.reshape(NCORE, NSUB, EPT)
        a = jnp.concatenate([a, spread], axis=2)
        return a.reshape(NCORE, NSUB, NBATCH, BATCH, CHUNK)

    def _prep_w(a):
        a = a.reshape(NCORE, NSUB, EPT)
        a = jnp.pad(a, ((0, 0), (0, 0), (0, pad)))
        return a.reshape(NCORE, NSUB, NBATCH, BATCH, CHUNK)

    src2 = _prep_idx(jnp.stack([edge_index_low[0], edge_index_nd_low[0]]))
    dst2 = _prep_idx(jnp.stack([edge_index_low[1], edge_index_nd_low[1]]))
    w2 = _prep_w(jnp.stack([edge_weight_low, edge_weight_nd_low]))
    zeros = jnp.zeros((N, D), jnp.float32)

    agg = _sc_aggregate(x, src2, dst2, w2, zeros)
    return _linear(agg[0], agg[1], W_conv, W_hiconv, W_lin,
                   b_conv.reshape(1, D), b_hiconv.reshape(1, D),
                   b_lin.reshape(1, D))
